# baseline (speedup 1.0000x reference)
import functools

import jax
import jax.numpy as jnp
from jax import lax
from jax.experimental import pallas as pl
from jax.experimental.pallas import tpu as pltpu
from jax.experimental.pallas import tpu_sc as plsc

_CHUNK = 400
_NBUF = 4


@jax.jit
def _gather_rows(idx_grouped, table):
    nw, nchunks, _ = idx_grouped.shape
    n = nw * nchunks * _CHUNK
    d = table.shape[1]
    per_worker = nchunks * _CHUNK
    nouter = nchunks // _NBUF
    info = plsc.get_sparse_core_info()
    assert nw == info.num_cores * info.num_subcores
    mesh = plsc.VectorSubcoreMesh(core_axis_name="c", subcore_axis_name="s")

    scratch = (
        [pltpu.VMEM((nchunks, _CHUNK), jnp.int32)]
        + [pltpu.VMEM((_CHUNK, d), jnp.float32) for _ in range(_NBUF)]
        + [pltpu.SemaphoreType.DMA for _ in range(2 * _NBUF)]
    )

    @functools.partial(
        pl.kernel,
        mesh=mesh,
        out_type=jax.ShapeDtypeStruct((n, d), jnp.float32),
        scratch_types=scratch,
        compiler_params=pltpu.CompilerParams(use_tc_tiling_on_sc=False),
    )
    def k(idx_hbm, table_hbm, out_hbm, idx_v, *bufs_and_sems):
        rows = bufs_and_sems[:_NBUF]
        gsem = bufs_and_sems[_NBUF:2 * _NBUF]
        ssem = bufs_and_sems[2 * _NBUF:]
        wid = lax.axis_index("s") * info.num_cores + lax.axis_index("c")
        base0 = wid * per_worker

        pltpu.sync_copy(idx_hbm.at[wid], idx_v)

        def gather_start(c, b):
            pltpu.async_copy(table_hbm.at[idx_v.at[c]], rows[b], gsem[b])

        def gather_wait(b):
            pltpu.make_async_copy(table_hbm.at[idx_v.at[0]], rows[b],
                                  gsem[b]).wait()

        def store_start(c, b):
            pltpu.async_copy(rows[b], out_hbm.at[pl.ds(base0 + c * _CHUNK,
                                                       _CHUNK)], ssem[b])

        def store_wait(b):
            pltpu.make_async_copy(rows[b], out_hbm.at[pl.ds(base0, _CHUNK)],
                                  ssem[b]).wait()

        for b in range(_NBUF):
            gather_start(b, b)

        def body(g, carry):
            for b in range(_NBUF):
                c = g * _NBUF + b
                gather_wait(b)
                store_start(c, b)
                nxt = c + _NBUF

                @pl.when(nxt < nchunks)
                def _():
                    store_wait(b)
                    gather_start(nxt, b)

            return carry

        lax.fori_loop(0, nouter, body, 0)
        for b in range(_NBUF):
            store_wait(b)

    return k(idx_grouped, table)


def kernel(indices, table):
    b, s = indices.shape
    d = table.shape[1]
    n = b * s
    info = plsc.get_sparse_core_info()
    nw = info.num_cores * info.num_subcores
    idx_grouped = indices.reshape(nw, n // (nw * _CHUNK), _CHUNK)
    out = _gather_rows(idx_grouped, table)
    return out.reshape(b, s, d)
